# baseline (speedup 1.0000x reference)
import jax
import jax.numpy as jnp
from jax import lax
from jax.experimental import pallas as pl
from jax.experimental.pallas import tpu as pltpu
from jax.experimental.pallas import tpu_sc as plsc

N_CODES = 8192
EMBED_DIM = 64
BETA = 0.25
ROWS = 8192
BLK = 256
NBLK = ROWS // BLK
EPS = 1e-12
DPAD = 128


def _tc_body(z_ref, embT_ref, prob_ref, zn_ref, idx_ref, csum_ref,
             cbT_ref, c2_ref):
    i = pl.program_id(0)

    @pl.when(i == 0)
    def _prologue():
        eT = embT_ref[...]
        n = jnp.sqrt(jnp.sum(eT * eT, axis=0, keepdims=True))
        cbT = eT / jnp.maximum(n, EPS)
        cbT_ref[...] = cbT
        c2_ref[...] = jnp.sum(cbT * cbT, axis=0, keepdims=True)
        csum_ref[...] = jnp.sum(jnp.abs(eT)).reshape(1, 1)

    zb = z_ref[...]
    zn = zb / jnp.maximum(
        jnp.sqrt(jnp.sum(zb * zb, axis=1, keepdims=True)), EPS)
    zn_ref[...] = jnp.concatenate(
        [zn, jnp.zeros((BLK, DPAD - EMBED_DIM), jnp.float32)], axis=1)
    s2 = lax.dot_general(zn + zn, cbT_ref[...], (((1,), (0,)), ((), ())),
                         preferred_element_type=jnp.float32)
    zi2 = jnp.sum(zn * zn, axis=1, keepdims=True)
    d = (zi2 + c2_ref[...]) - s2
    LW = 128
    NCH = N_CODES // LW
    vals = [d[:, c * LW:(c + 1) * LW] for c in range(NCH)]
    nv, ni = [], []
    for p in range(0, NCH, 2):
        a, b = vals[p], vals[p + 1]
        lt = b < a
        nv.append(jnp.minimum(a, b))
        ni.append(jnp.where(lt, jnp.int32(p + 1), jnp.int32(p)))
    vals, idxs = nv, ni
    while len(vals) > 1:
        nv, ni = [], []
        for p in range(0, len(vals), 2):
            a, b = vals[p], vals[p + 1]
            lt = b < a
            nv.append(jnp.minimum(a, b))
            ni.append(jnp.where(lt, idxs[p + 1], idxs[p]))
        vals, idxs = nv, ni
    finval, finchk = vals[0], idxs[0]
    dmin = jnp.min(finval, axis=1, keepdims=True)
    ex = jnp.exp(dmin - d)
    prob_ref[...] = ex / jnp.sum(ex, axis=1, keepdims=True)
    lane = lax.broadcasted_iota(jnp.int32, (BLK, LW), 1)
    gidx = finchk * LW + lane
    idx = jnp.min(jnp.where(finval == dmin, gidx, jnp.int32(N_CODES)), axis=1)
    idx_ref[0, 0, :] = idx


def _tc_stage(z_flat, embT):
    grid = (NBLK,)
    return pl.pallas_call(
        _tc_body,
        grid=grid,
        in_specs=[
            pl.BlockSpec((BLK, EMBED_DIM), lambda i: (i, 0)),
            pl.BlockSpec((EMBED_DIM, N_CODES), lambda i: (0, 0)),
        ],
        out_specs=[
            pl.BlockSpec((BLK, N_CODES), lambda i: (i, 0)),
            pl.BlockSpec((BLK, DPAD), lambda i: (i, 0)),
            pl.BlockSpec((1, 1, BLK), lambda i: (i, 0, 0)),
            pl.BlockSpec((1, 1), lambda i: (0, 0)),
        ],
        out_shape=[
            jax.ShapeDtypeStruct((ROWS, N_CODES), jnp.float32),
            jax.ShapeDtypeStruct((ROWS, DPAD), jnp.float32),
            jax.ShapeDtypeStruct((NBLK, 1, BLK), jnp.int32),
            jax.ShapeDtypeStruct((1, 1), jnp.float32),
        ],
        scratch_shapes=[
            pltpu.VMEM((EMBED_DIM, N_CODES), jnp.float32),
            pltpu.VMEM((1, N_CODES), jnp.float32),
        ],
    )(z_flat, embT)


_SC_CHUNK = 128
_NW = 32
_BPW = ROWS // _NW
_VPR = DPAD // 16


def _sc_body(zn_hbm, idx_hbm, emb_hbm, part_hbm,
             idx_a, idx_b, rows_a, rows_b, za, zb, part_v,
             sem_i, sem_z, sem_g, sem_o):
    wid = lax.axis_index("s") * 2 + lax.axis_index("c")
    base = wid * _BPW
    ci1 = pltpu.async_copy(idx_hbm.at[pl.ds(base, _SC_CHUNK)], idx_a, sem_i)
    ci2 = pltpu.async_copy(
        idx_hbm.at[pl.ds(base + _SC_CHUNK, _SC_CHUNK)], idx_b, sem_i)
    cz1 = pltpu.async_copy(zn_hbm.at[pl.ds(base, _SC_CHUNK)], za, sem_z)
    cz2 = pltpu.async_copy(
        zn_hbm.at[pl.ds(base + _SC_CHUNK, _SC_CHUNK)], zb, sem_z)
    ci1.wait()
    ci2.wait()
    ca = pltpu.async_copy(zn_hbm.at[idx_a], rows_a, sem_g)
    cb = pltpu.async_copy(zn_hbm.at[idx_b], rows_b, sem_g)
    cz1.wait()
    cz2.wait()
    ca.wait()
    cb.wait()
    co1 = pltpu.async_copy(rows_a, emb_hbm.at[pl.ds(base, _SC_CHUNK)], sem_o)
    co2 = pltpu.async_copy(
        rows_b, emb_hbm.at[pl.ds(base + _SC_CHUNK, _SC_CHUNK)], sem_o)

    def row_sum(r, acc, zv, gv):
        for c in range(_VPR):
            dlt = zv[r, pl.ds(c * 16, 16)] - gv[r, pl.ds(c * 16, 16)]
            acc = acc + dlt * dlt
        return acc

    acc = jnp.zeros((16,), jnp.float32)
    acc = lax.fori_loop(
        0, _SC_CHUNK, lambda r, a: row_sum(r, a, za, rows_a), acc)
    acc = lax.fori_loop(
        0, _SC_CHUNK, lambda r, a: row_sum(r, a, zb, rows_b), acc)
    part_v[...] = acc
    co1.wait()
    co2.wait()
    pltpu.sync_copy(part_v, part_hbm.at[wid])


def _sc_stage(zn, idx):
    mesh = plsc.VectorSubcoreMesh(core_axis_name="c", subcore_axis_name="s")
    return pl.kernel(
        _sc_body,
        out_type=[
            jax.ShapeDtypeStruct((ROWS, DPAD), jnp.float32),
            jax.ShapeDtypeStruct((_NW, 16), jnp.float32),
        ],
        mesh=mesh,
        scratch_types=[
            pltpu.VMEM((_SC_CHUNK,), jnp.int32),
            pltpu.VMEM((_SC_CHUNK,), jnp.int32),
            pltpu.VMEM((_SC_CHUNK, DPAD), jnp.float32),
            pltpu.VMEM((_SC_CHUNK, DPAD), jnp.float32),
            pltpu.VMEM((_SC_CHUNK, DPAD), jnp.float32),
            pltpu.VMEM((_SC_CHUNK, DPAD), jnp.float32),
            pltpu.VMEM((16,), jnp.float32),
            pltpu.SemaphoreType.DMA,
            pltpu.SemaphoreType.DMA,
            pltpu.SemaphoreType.DMA,
            pltpu.SemaphoreType.DMA,
        ],
    )(zn, idx)


def kernel(z, embeddings):
    b, d, h, w = z.shape
    z_flat = jnp.transpose(z, (0, 2, 3, 1)).reshape(-1, d)
    prob, zn, idx, csum = _tc_stage(z_flat, embeddings.T)
    emb, parts = _sc_stage(zn, idx.reshape(ROWS))
    commitment = jnp.sum(parts) / jnp.float32(ROWS * EMBED_DIM)
    loss = jnp.float32(BETA) * commitment
    q = jnp.transpose(emb[:, :EMBED_DIM].reshape(b, h, w, d), (0, 3, 1, 2))
    return (q, commitment, loss, csum.reshape(()), prob)


# speedup vs baseline: 1.0237x; 1.0237x over previous
import jax
import jax.numpy as jnp
from jax import lax
from jax.experimental import pallas as pl
from jax.experimental.pallas import tpu as pltpu
from jax.experimental.pallas import tpu_sc as plsc

N_CODES = 8192
EMBED_DIM = 64
BETA = 0.25
ROWS = 8192
BLK = 512
NBLK = ROWS // BLK
EPS = 1e-12
DPAD = 128


def _tc_body(z_ref, embT_ref, prob_ref, zn_ref, idx_ref, csum_ref,
             cbT_ref, c2_ref):
    i = pl.program_id(0)

    @pl.when(i == 0)
    def _prologue():
        eT = embT_ref[...]
        n = jnp.sqrt(jnp.sum(eT * eT, axis=0, keepdims=True))
        cbT = eT / jnp.maximum(n, EPS)
        cbT_ref[...] = cbT
        c2_ref[...] = jnp.sum(cbT * cbT, axis=0, keepdims=True)
        csum_ref[...] = jnp.sum(jnp.abs(eT)).reshape(1, 1)

    zb = z_ref[...]
    zn = zb / jnp.maximum(
        jnp.sqrt(jnp.sum(zb * zb, axis=1, keepdims=True)), EPS)
    zn_ref[...] = jnp.concatenate(
        [zn, jnp.zeros((BLK, DPAD - EMBED_DIM), jnp.float32)], axis=1)
    s2 = lax.dot_general(zn + zn, cbT_ref[...], (((1,), (0,)), ((), ())),
                         preferred_element_type=jnp.float32)
    zi2 = jnp.sum(zn * zn, axis=1, keepdims=True)
    d = (zi2 + c2_ref[...]) - s2
    LW = 128
    NCH = N_CODES // LW
    vals = [d[:, c * LW:(c + 1) * LW] for c in range(NCH)]
    nv, ni = [], []
    for p in range(0, NCH, 2):
        a, b = vals[p], vals[p + 1]
        lt = b < a
        nv.append(jnp.minimum(a, b))
        ni.append(jnp.where(lt, jnp.int32(p + 1), jnp.int32(p)))
    vals, idxs = nv, ni
    while len(vals) > 1:
        nv, ni = [], []
        for p in range(0, len(vals), 2):
            a, b = vals[p], vals[p + 1]
            lt = b < a
            nv.append(jnp.minimum(a, b))
            ni.append(jnp.where(lt, idxs[p + 1], idxs[p]))
        vals, idxs = nv, ni
    finval, finchk = vals[0], idxs[0]
    dmin = jnp.min(finval, axis=1, keepdims=True)
    ex = jnp.exp(dmin - d)
    prob_ref[...] = ex / jnp.sum(ex, axis=1, keepdims=True)
    lane = lax.broadcasted_iota(jnp.int32, (BLK, LW), 1)
    gidx = finchk * LW + lane
    idx = jnp.min(jnp.where(finval == dmin, gidx, jnp.int32(N_CODES)), axis=1)
    idx_ref[0, 0, :] = idx


def _tc_stage(z_flat, embT):
    grid = (NBLK,)
    return pl.pallas_call(
        _tc_body,
        grid=grid,
        in_specs=[
            pl.BlockSpec((BLK, EMBED_DIM), lambda i: (i, 0)),
            pl.BlockSpec((EMBED_DIM, N_CODES), lambda i: (0, 0)),
        ],
        out_specs=[
            pl.BlockSpec((BLK, N_CODES), lambda i: (i, 0)),
            pl.BlockSpec((BLK, DPAD), lambda i: (i, 0)),
            pl.BlockSpec((1, 1, BLK), lambda i: (i, 0, 0)),
            pl.BlockSpec((1, 1), lambda i: (0, 0)),
        ],
        out_shape=[
            jax.ShapeDtypeStruct((ROWS, N_CODES), jnp.float32),
            jax.ShapeDtypeStruct((ROWS, DPAD), jnp.float32),
            jax.ShapeDtypeStruct((NBLK, 1, BLK), jnp.int32),
            jax.ShapeDtypeStruct((1, 1), jnp.float32),
        ],
        scratch_shapes=[
            pltpu.VMEM((EMBED_DIM, N_CODES), jnp.float32),
            pltpu.VMEM((1, N_CODES), jnp.float32),
        ],
        compiler_params=pltpu.CompilerParams(
            vmem_limit_bytes=100 * 1024 * 1024),
    )(z_flat, embT)


_SC_CHUNK = 128
_NW = 32
_BPW = ROWS // _NW
_VPR = DPAD // 16


def _sc_body(zn_hbm, idx_hbm, emb_hbm, part_hbm,
             idx_a, idx_b, rows_a, rows_b, za, zb, part_v,
             sem_i, sem_z, sem_g, sem_o):
    wid = lax.axis_index("s") * 2 + lax.axis_index("c")
    base = wid * _BPW
    ci1 = pltpu.async_copy(idx_hbm.at[pl.ds(base, _SC_CHUNK)], idx_a, sem_i)
    ci2 = pltpu.async_copy(
        idx_hbm.at[pl.ds(base + _SC_CHUNK, _SC_CHUNK)], idx_b, sem_i)
    cz1 = pltpu.async_copy(zn_hbm.at[pl.ds(base, _SC_CHUNK)], za, sem_z)
    cz2 = pltpu.async_copy(
        zn_hbm.at[pl.ds(base + _SC_CHUNK, _SC_CHUNK)], zb, sem_z)
    ci1.wait()
    ci2.wait()
    ca = pltpu.async_copy(zn_hbm.at[idx_a], rows_a, sem_g)
    cb = pltpu.async_copy(zn_hbm.at[idx_b], rows_b, sem_g)
    cz1.wait()
    cz2.wait()
    ca.wait()
    cb.wait()
    co1 = pltpu.async_copy(rows_a, emb_hbm.at[pl.ds(base, _SC_CHUNK)], sem_o)
    co2 = pltpu.async_copy(
        rows_b, emb_hbm.at[pl.ds(base + _SC_CHUNK, _SC_CHUNK)], sem_o)

    def row_sum(r, acc, zv, gv):
        for c in range(_VPR):
            dlt = zv[r, pl.ds(c * 16, 16)] - gv[r, pl.ds(c * 16, 16)]
            acc = acc + dlt * dlt
        return acc

    acc = jnp.zeros((16,), jnp.float32)
    acc = lax.fori_loop(
        0, _SC_CHUNK, lambda r, a: row_sum(r, a, za, rows_a), acc)
    acc = lax.fori_loop(
        0, _SC_CHUNK, lambda r, a: row_sum(r, a, zb, rows_b), acc)
    part_v[...] = acc
    co1.wait()
    co2.wait()
    pltpu.sync_copy(part_v, part_hbm.at[wid])


def _sc_stage(zn, idx):
    mesh = plsc.VectorSubcoreMesh(core_axis_name="c", subcore_axis_name="s")
    return pl.kernel(
        _sc_body,
        out_type=[
            jax.ShapeDtypeStruct((ROWS, DPAD), jnp.float32),
            jax.ShapeDtypeStruct((_NW, 16), jnp.float32),
        ],
        mesh=mesh,
        scratch_types=[
            pltpu.VMEM((_SC_CHUNK,), jnp.int32),
            pltpu.VMEM((_SC_CHUNK,), jnp.int32),
            pltpu.VMEM((_SC_CHUNK, DPAD), jnp.float32),
            pltpu.VMEM((_SC_CHUNK, DPAD), jnp.float32),
            pltpu.VMEM((_SC_CHUNK, DPAD), jnp.float32),
            pltpu.VMEM((_SC_CHUNK, DPAD), jnp.float32),
            pltpu.VMEM((16,), jnp.float32),
            pltpu.SemaphoreType.DMA,
            pltpu.SemaphoreType.DMA,
            pltpu.SemaphoreType.DMA,
            pltpu.SemaphoreType.DMA,
        ],
    )(zn, idx)


def kernel(z, embeddings):
    b, d, h, w = z.shape
    z_flat = jnp.transpose(z, (0, 2, 3, 1)).reshape(-1, d)
    prob, zn, idx, csum = _tc_stage(z_flat, embeddings.T)
    emb, parts = _sc_stage(zn, idx.reshape(ROWS))
    commitment = jnp.sum(parts) / jnp.float32(ROWS * EMBED_DIM)
    loss = jnp.float32(BETA) * commitment
    q = jnp.transpose(emb[:, :EMBED_DIM].reshape(b, h, w, d), (0, 3, 1, 2))
    return (q, commitment, loss, csum.reshape(()), prob)
